# baseline (speedup 1.0000x reference)
import jax
import jax.numpy as jnp
from jax import lax
from jax.experimental import pallas as pl
from jax.experimental.pallas import tpu as pltpu
from jax.experimental.pallas import tpu_sc as plsc

N = 10000
E = 320000
D = 128
DA = D + 16
NC = 2
NS = 16
NW = NC * NS
BLK = 128
CHUNK = 8
NCHUNK = -(-E // (NW * BLK * CHUNK))
ITERS = NCHUNK * CHUNK
E_PAD = NW * BLK * ITERS
NPAD = 10112
RPT = NPAD // NS


def _sc_body(ha_hbm, ei_hbm, zeros_hbm, out, sh, idxbuf, rows0, rows1, sem0, sem1):
    c = lax.axis_index("c")
    s = lax.axis_index("s")
    wid = s * NC + c
    r0 = s * RPT

    pltpu.sync_copy(zeros_hbm.at[pl.ds(r0, RPT)], sh.at[pl.ds(r0, RPT)])
    plsc.subcore_barrier()

    rows = (rows0, rows1)
    sems = (sem0, sem1)

    @pl.loop(0, NCHUNK)
    def _chunk(q):
        @pl.when(q > 0)
        def _scatter_tail():
            pltpu.sync_copy(rows[(CHUNK - 1) % 2],
                            sh.at[idxbuf.at[2 * CHUNK - 1]], add=True)
        pltpu.sync_copy(ei_hbm.at[wid].at[q], idxbuf)

        for k in range(CHUNK):
            desc = pltpu.async_copy(ha_hbm.at[idxbuf.at[2 * k]], rows[k % 2],
                                    sems[k % 2])
            if k > 0:
                pltpu.sync_copy(rows[(k - 1) % 2],
                                sh.at[idxbuf.at[2 * k - 1]], add=True)
            desc.wait()

    pltpu.sync_copy(rows[(CHUNK - 1) % 2],
                    sh.at[idxbuf.at[2 * CHUNK - 1]], add=True)

    plsc.subcore_barrier()
    pltpu.sync_copy(sh.at[pl.ds(r0, RPT)], out.at[c].at[pl.ds(r0, RPT)])


def _sc_aggregate(h_aug, ei, zeros):
    mesh = plsc.VectorSubcoreMesh(core_axis_name="c", subcore_axis_name="s",
                                  num_cores=NC, num_subcores=NS)
    return pl.kernel(
        _sc_body,
        out_type=jax.ShapeDtypeStruct((NC, NPAD, DA), jnp.float32),
        mesh=mesh,
        compiler_params=pltpu.CompilerParams(use_tc_tiling_on_sc=False),
        scratch_types=[
            pltpu.VMEM_SHARED((NPAD, DA), jnp.float32),
            pltpu.VMEM((2 * CHUNK, BLK), jnp.int32),
            pltpu.VMEM((BLK, DA), jnp.float32),
            pltpu.VMEM((BLK, DA), jnp.float32),
            pltpu.SemaphoreType.DMA,
            pltpu.SemaphoreType.DMA,
        ],
    )(h_aug, ei, zeros)


ROWS_TC = 400


def _tc_body(h, a0, a1, ws, wn, bs, bn, g, b, o):
    deg = jnp.maximum(a0[:, D:D + 1] + a1[:, D:D + 1], 1.0)
    neigh = (a0[:, :D] + a1[:, :D]) / deg
    dn = (((1,), (1,)), ((), ()))
    z = (lax.dot_general(h[...], ws[...], dn, preferred_element_type=jnp.float32)
         + bs[...]
         + lax.dot_general(neigh, wn[...], dn, preferred_element_type=jnp.float32)
         + bn[...])
    z = z * jax.nn.sigmoid(z)
    mu = jnp.mean(z, axis=-1, keepdims=True)
    r = z - mu
    var = jnp.mean(r * r, axis=-1, keepdims=True)
    o[...] = r * lax.rsqrt(var + 1e-5) * g[...] + b[...]


def _tc_dense(h, a0, a1, W_self, W_neigh, b_self, b_neigh, ln_g, ln_b):
    grid = (N // ROWS_TC,)
    row_spec = pl.BlockSpec((ROWS_TC, D), lambda i: (i, 0))
    acc_spec = pl.BlockSpec((ROWS_TC, DA), lambda i: (i, 0))
    w_spec = pl.BlockSpec((D, D), lambda i: (0, 0))
    v_spec = pl.BlockSpec((1, D), lambda i: (0, 0))
    return pl.pallas_call(
        _tc_body,
        grid=grid,
        in_specs=[row_spec, acc_spec, acc_spec,
                  w_spec, w_spec, v_spec, v_spec, v_spec, v_spec],
        out_specs=row_spec,
        out_shape=jax.ShapeDtypeStruct((N, D), jnp.float32),
    )(h, a0, a1, W_self, W_neigh,
      b_self.reshape(1, D), b_neigh.reshape(1, D),
      ln_g.reshape(1, D), ln_b.reshape(1, D))


def kernel(h, edge_index, W_self, b_self, W_neigh, b_neigh, ln_g, ln_b):
    pad = E_PAD - E
    src4 = jnp.concatenate([edge_index[0], jnp.zeros((pad,), jnp.int32)]
                           ).reshape(NW, NCHUNK, CHUNK, 1, BLK)
    dst4 = jnp.concatenate([edge_index[1], jnp.full((pad,), N, jnp.int32)]
                           ).reshape(NW, NCHUNK, CHUNK, 1, BLK)
    ei = jnp.concatenate([src4, dst4], axis=3).reshape(NW, NCHUNK, 2 * CHUNK, BLK)
    h_aug = jnp.concatenate(
        [h, jnp.ones((N, 1), jnp.float32), jnp.zeros((N, DA - D - 1), jnp.float32)],
        axis=1)
    zeros = jnp.zeros((NPAD, DA), jnp.float32)
    acc = _sc_aggregate(h_aug, ei, zeros)
    return _tc_dense(h, acc[0, :N], acc[1, :N],
                     W_self, W_neigh, b_self, b_neigh, ln_g, ln_b)


# speedup vs baseline: 1.0774x; 1.0774x over previous
import jax
import jax.numpy as jnp
from jax import lax
from jax.experimental import pallas as pl
from jax.experimental.pallas import tpu as pltpu
from jax.experimental.pallas import tpu_sc as plsc

N = 10000
E = 320000
D = 128
DA = D + 16
NC = 2
NS = 16
NW = NC * NS
BLK = 256
NSUP = -(-E // (NW * BLK))
ITERS = NSUP
E_PAD = NW * BLK * ITERS
NPAD = 10112
RPT = NPAD // NS


def _sc_body(ha_hbm, ei_hbm, zeros_hbm, out, sh, idxbuf, rows, sem):
    c = lax.axis_index("c")
    s = lax.axis_index("s")
    wid = s * NC + c
    r0 = s * RPT

    pltpu.sync_copy(zeros_hbm.at[pl.ds(r0, RPT)], sh.at[pl.ds(r0, RPT)])
    plsc.subcore_barrier()

    @pl.loop(0, NSUP)
    def _super(p):
        pltpu.sync_copy(ei_hbm.at[wid].at[p], idxbuf)
        pltpu.async_copy(ha_hbm.at[idxbuf.at[0]], rows, sem).wait()
        pltpu.sync_copy(rows, sh.at[idxbuf.at[1]], add=True)

    plsc.subcore_barrier()
    pltpu.sync_copy(sh.at[pl.ds(r0, RPT)], out.at[c].at[pl.ds(r0, RPT)])


def _sc_aggregate(h_aug, ei, zeros):
    mesh = plsc.VectorSubcoreMesh(core_axis_name="c", subcore_axis_name="s",
                                  num_cores=NC, num_subcores=NS)
    return pl.kernel(
        _sc_body,
        out_type=jax.ShapeDtypeStruct((NC, NPAD, DA), jnp.float32),
        mesh=mesh,
        compiler_params=pltpu.CompilerParams(use_tc_tiling_on_sc=False),
        scratch_types=[
            pltpu.VMEM_SHARED((NPAD, DA), jnp.float32),
            pltpu.VMEM((2, BLK), jnp.int32),
            pltpu.VMEM((BLK, DA), jnp.float32),
            pltpu.SemaphoreType.DMA,
        ],
    )(h_aug, ei, zeros)


ROWS_TC = 400


def _tc_body(h, a0, a1, ws, wn, bs, bn, g, b, o):
    a0 = a0[0]
    a1 = a1[0]
    deg = jnp.maximum(a0[:, D:D + 1] + a1[:, D:D + 1], 1.0)
    neigh = (a0[:, :D] + a1[:, :D]) / deg
    dn = (((1,), (1,)), ((), ()))
    z = (lax.dot_general(h[...], ws[...], dn, preferred_element_type=jnp.float32)
         + bs[...]
         + lax.dot_general(neigh, wn[...], dn, preferred_element_type=jnp.float32)
         + bn[...])
    z = z * jax.nn.sigmoid(z)
    mu = jnp.mean(z, axis=-1, keepdims=True)
    r = z - mu
    var = jnp.mean(r * r, axis=-1, keepdims=True)
    o[...] = r * lax.rsqrt(var + 1e-5) * g[...] + b[...]


def _tc_dense(h, acc, W_self, W_neigh, b_self, b_neigh, ln_g, ln_b):
    grid = (N // ROWS_TC,)
    row_spec = pl.BlockSpec((ROWS_TC, D), lambda i: (i, 0))
    acc0_spec = pl.BlockSpec((1, ROWS_TC, DA), lambda i: (0, i, 0))
    acc1_spec = pl.BlockSpec((1, ROWS_TC, DA), lambda i: (1, i, 0))
    w_spec = pl.BlockSpec((D, D), lambda i: (0, 0))
    v_spec = pl.BlockSpec((1, D), lambda i: (0, 0))
    return pl.pallas_call(
        _tc_body,
        grid=grid,
        in_specs=[row_spec, acc0_spec, acc1_spec,
                  w_spec, w_spec, v_spec, v_spec, v_spec, v_spec],
        out_specs=row_spec,
        out_shape=jax.ShapeDtypeStruct((N, D), jnp.float32),
    )(h, acc, acc, W_self, W_neigh,
      b_self.reshape(1, D), b_neigh.reshape(1, D),
      ln_g.reshape(1, D), ln_b.reshape(1, D))


def kernel(h, edge_index, W_self, b_self, W_neigh, b_neigh, ln_g, ln_b):
    pad = E_PAD - E
    src4 = jnp.concatenate([edge_index[0], jnp.zeros((pad,), jnp.int32)]
                           ).reshape(NW, NSUP, 1, BLK)
    dst4 = jnp.concatenate([edge_index[1], jnp.full((pad,), N, jnp.int32)]
                           ).reshape(NW, NSUP, 1, BLK)
    ei = jnp.concatenate([src4, dst4], axis=2)
    h_aug = jnp.concatenate(
        [h, jnp.ones((N, 1), jnp.float32), jnp.zeros((N, DA - D - 1), jnp.float32)],
        axis=1)
    zeros = jnp.zeros((NPAD, DA), jnp.float32)
    acc = _sc_aggregate(h_aug, ei, zeros)
    return _tc_dense(h, acc, W_self, W_neigh, b_self, b_neigh, ln_g, ln_b)
